# initial kernel scaffold (unmeasured)
import jax
import jax.numpy as jnp
from jax import lax
from jax.experimental import pallas as pl
from jax.experimental.pallas import tpu as pltpu

N_DEV = 16


def _gelu(z):
    return 0.5 * z * (1.0 + jnp.tanh(0.7978845608 * (z + 0.044715 * z * z * z)))


def kernel(A, B):
    m, k = A.shape
    k2, n = B.shape
    assert k == k2
    m_chunk = m // N_DEV

    def body(a_ref, b_ref, out_ref, send_buf, recv_buf, send_sems, recv_sems,
             credit_sem):
        my = lax.axis_index("i")
        left = lax.rem(my + N_DEV - 1, N_DEV)
        right = lax.rem(my + 1, N_DEV)

        barrier_sem = pltpu.get_barrier_semaphore()
        for nbr in (left, right):
            pl.semaphore_signal(
                barrier_sem, inc=1,
                device_id=(nbr,), device_id_type=pl.DeviceIdType.MESH,
            )
        pl.semaphore_wait(barrier_sem, 2)

        out_ref[:, :] = jnp.dot(
            a_ref[:, :], b_ref[:, :], preferred_element_type=jnp.float32
        )

        def rows(j):
            return pl.ds(j * m_chunk, m_chunk)

        def hop(h, get_send_data):
            slot = h % 2
            send_buf[slot, :, :] = get_send_data()
            if h >= 2:
                pl.semaphore_wait(credit_sem, 1)
            rdma = pltpu.make_async_remote_copy(
                src_ref=send_buf.at[slot],
                dst_ref=recv_buf.at[slot],
                send_sem=send_sems.at[slot],
                recv_sem=recv_sems.at[slot],
                device_id=(right,),
                device_id_type=pl.DeviceIdType.MESH,
            )
            rdma.start()
            rdma.wait()
            pl.semaphore_signal(
                credit_sem, inc=1,
                device_id=(left,), device_id_type=pl.DeviceIdType.MESH,
            )

        for s in range(N_DEV - 1):
            j_send = lax.rem(my + N_DEV - s, N_DEV)
            if s == 0:
                hop(s, lambda: out_ref[rows(j_send), :])
            else:
                prev = (s - 1) % 2
                j_prev = lax.rem(my + N_DEV - s, N_DEV)
                hop(s, lambda: recv_buf[prev, :, :] + out_ref[rows(j_prev), :])

        j_own = lax.rem(my + 1, N_DEV)
        reduced = recv_buf[(N_DEV - 2) % 2, :, :] + out_ref[rows(j_own), :]
        out_ref[rows(j_own), :] = _gelu(reduced)

        for s in range(N_DEV - 1):
            h = (N_DEV - 1) + s
            if s == 0:
                hop(h, lambda: out_ref[rows(j_own), :])
            else:
                prev = (h - 1) % 2
                hop(h, lambda: recv_buf[prev, :, :])
            j_recv = lax.rem(my + N_DEV - s, N_DEV)
            out_ref[rows(j_recv), :] = recv_buf[h % 2, :, :]

        pl.semaphore_wait(credit_sem, 2)

    return pl.pallas_call(
        body,
        out_shape=jax.ShapeDtypeStruct((m, n), jnp.float32),
        in_specs=[
            pl.BlockSpec(memory_space=pltpu.VMEM),
            pl.BlockSpec(memory_space=pltpu.VMEM),
        ],
        out_specs=pl.BlockSpec(memory_space=pltpu.VMEM),
        scratch_shapes=[
            pltpu.VMEM((2, m // N_DEV, n), jnp.float32),
            pltpu.VMEM((2, m // N_DEV, n), jnp.float32),
            pltpu.SemaphoreType.DMA((2,)),
            pltpu.SemaphoreType.DMA((2,)),
            pltpu.SemaphoreType.REGULAR,
        ],
        compiler_params=pltpu.CompilerParams(collective_id=0),
    )(A, B)


# baseline (device time: 262267 ns/iter reference)
import jax
import jax.numpy as jnp
from jax import lax
from jax.experimental import pallas as pl
from jax.experimental.pallas import tpu as pltpu

N_DEV = 16


def _gelu(z):
    return 0.5 * z * (1.0 + jnp.tanh(0.7978845608 * (z + 0.044715 * z * z * z)))


def kernel(A, B):
    m, k = A.shape
    k2, n = B.shape
    assert k == k2
    m_chunk = m // N_DEV

    def body(a_ref, b_ref, out_ref, send_buf, recv_buf, send_sems, recv_sems,
             credit_sem):
        my = lax.axis_index("i")
        left = lax.rem(my + N_DEV - 1, N_DEV)
        right = lax.rem(my + 1, N_DEV)

        barrier_sem = pltpu.get_barrier_semaphore()
        for nbr in (left, right):
            pl.semaphore_signal(
                barrier_sem, inc=1,
                device_id=(nbr,), device_id_type=pl.DeviceIdType.MESH,
            )
        pl.semaphore_wait(barrier_sem, 2)

        out_ref[:, :] = jnp.dot(
            a_ref[:, :], b_ref[:, :], preferred_element_type=jnp.float32
        )

        def rows(j):
            return pl.ds(j * m_chunk, m_chunk)

        def credit_left():
            pl.semaphore_signal(
                credit_sem, inc=1,
                device_id=(left,), device_id_type=pl.DeviceIdType.MESH,
            )

        def send_hop(h):
            if h >= 2:
                pl.semaphore_wait(credit_sem, 1)
            rdma = pltpu.make_async_remote_copy(
                src_ref=send_buf.at[h % 2],
                dst_ref=recv_buf.at[h % 2],
                send_sem=send_sems.at[h % 2],
                recv_sem=recv_sems.at[h % 2],
                device_id=(right,),
                device_id_type=pl.DeviceIdType.MESH,
            )
            rdma.start()
            rdma.wait()

        for s in range(N_DEV - 1):
            j_send = lax.rem(my + N_DEV - s, N_DEV)
            if s == 0:
                send_buf[s % 2, :, :] = out_ref[rows(j_send), :]
            else:
                send_buf[s % 2, :, :] = (
                    recv_buf[(s - 1) % 2, :, :] + out_ref[rows(j_send), :]
                )
                credit_left()
            send_hop(s)

        j_own = lax.rem(my + 1, N_DEV)
        reduced = recv_buf[(N_DEV - 2) % 2, :, :] + out_ref[rows(j_own), :]
        out_ref[rows(j_own), :] = _gelu(reduced)
        credit_left()

        for s in range(N_DEV - 1):
            h = (N_DEV - 1) + s
            if s == 0:
                send_buf[h % 2, :, :] = out_ref[rows(j_own), :]
            else:
                send_buf[h % 2, :, :] = recv_buf[(h - 1) % 2, :, :]
                credit_left()
            send_hop(h)
            j_recv = lax.rem(my + N_DEV - s, N_DEV)
            out_ref[rows(j_recv), :] = recv_buf[h % 2, :, :]
        credit_left()

        pl.semaphore_wait(credit_sem, 2)

    return pl.pallas_call(
        body,
        out_shape=jax.ShapeDtypeStruct((m, n), jnp.float32),
        in_specs=[
            pl.BlockSpec(memory_space=pltpu.VMEM),
            pl.BlockSpec(memory_space=pltpu.VMEM),
        ],
        out_specs=pl.BlockSpec(memory_space=pltpu.VMEM),
        scratch_shapes=[
            pltpu.VMEM((2, m // N_DEV, n), jnp.float32),
            pltpu.VMEM((2, m // N_DEV, n), jnp.float32),
            pltpu.SemaphoreType.DMA((2,)),
            pltpu.SemaphoreType.DMA((2,)),
            pltpu.SemaphoreType.REGULAR,
        ],
        compiler_params=pltpu.CompilerParams(collective_id=0),
    )(A, B)


# device time: 139427 ns/iter; 1.8810x vs baseline; 1.8810x over previous
import jax
import jax.numpy as jnp
from jax import lax
from jax.experimental import pallas as pl
from jax.experimental.pallas import tpu as pltpu

N_DEV = 16
N_STREAMS = 4
N_HOPS = 2 * (N_DEV - 1)


def _gelu(z):
    return 0.5 * z * (1.0 + jnp.tanh(0.7978845608 * (z + 0.044715 * z * z * z)))


def kernel(A, B):
    m, k = A.shape
    k2, n = B.shape
    assert k == k2
    m_chunk = m // N_DEV
    n_stream = n // N_STREAMS

    def body(a_ref, b_ref, out_ref, send_buf, recv_buf, send_sems, recv_sems,
             credit_sems):
        my = lax.axis_index("i")
        left = lax.rem(my + N_DEV - 1, N_DEV)
        right = lax.rem(my + 1, N_DEV)

        barrier_sem = pltpu.get_barrier_semaphore()
        for nbr in (left, right):
            pl.semaphore_signal(
                barrier_sem, inc=1,
                device_id=(nbr,), device_id_type=pl.DeviceIdType.MESH,
            )
        pl.semaphore_wait(barrier_sem, 2)

        out_ref[:, :] = jnp.dot(
            a_ref[:, :], b_ref[:, :], preferred_element_type=jnp.float32
        )

        def rows(j):
            return pl.ds(j * m_chunk, m_chunk)

        def is_r(kk):
            return kk < 2

        def cols(kk):
            return pl.ds(kk * n_stream, n_stream)

        def dst(kk):
            return right if is_r(kk) else left

        def crd(kk):
            return left if is_r(kk) else right

        def rs_send_chunk(kk, s):
            if is_r(kk):
                return lax.rem(my + N_DEV - s, N_DEV)
            return lax.rem(my + s, N_DEV)

        def own_chunk(kk):
            if is_r(kk):
                return lax.rem(my + 1, N_DEV)
            return lax.rem(my + N_DEV - 1, N_DEV)

        def ag_recv_chunk(kk, s):
            if is_r(kk):
                return lax.rem(my + N_DEV - s, N_DEV)
            return lax.rem(my + s, N_DEV)

        def rdma(kk, h):
            slot = h % 2
            return pltpu.make_async_remote_copy(
                src_ref=send_buf.at[kk, slot],
                dst_ref=recv_buf.at[kk, slot],
                send_sem=send_sems.at[kk, slot],
                recv_sem=recv_sems.at[kk, slot],
                device_id=(dst(kk),),
                device_id_type=pl.DeviceIdType.MESH,
            )

        def credit(kk):
            pl.semaphore_signal(
                credit_sems.at[kk], inc=1,
                device_id=(crd(kk),), device_id_type=pl.DeviceIdType.MESH,
            )

        def stage(kk, h):
            slot = h % 2
            if h == 0:
                send_buf[kk, slot, :, :] = out_ref[rows(rs_send_chunk(kk, 0)),
                                                   cols(kk)]
            elif h <= N_DEV - 2:
                send_buf[kk, slot, :, :] = (
                    recv_buf[kk, (h - 1) % 2, :, :]
                    + out_ref[rows(rs_send_chunk(kk, h)), cols(kk)]
                )
            elif h == N_DEV - 1:
                red = (recv_buf[kk, (h - 1) % 2, :, :]
                       + out_ref[rows(own_chunk(kk)), cols(kk)])
                g = _gelu(red)
                out_ref[rows(own_chunk(kk)), cols(kk)] = g
                send_buf[kk, slot, :, :] = g
            else:
                send_buf[kk, slot, :, :] = recv_buf[kk, (h - 1) % 2, :, :]

        def store(kk, h):
            if h >= N_DEV - 1:
                s = h - (N_DEV - 1)
                out_ref[rows(ag_recv_chunk(kk, s)), cols(kk)] = \
                    recv_buf[kk, h % 2, :, :]

        order = (0, 2, 1, 3)

        for kk in order:
            stage(kk, 0)
            rdma(kk, 0).start()

        for h in range(1, N_HOPS):
            for kk in order:
                if h >= 2:
                    rdma(kk, h).wait_send()
                rdma(kk, h - 1).wait_recv()
                store(kk, h - 1)
                stage(kk, h)
                credit(kk)
                if h >= 2:
                    pl.semaphore_wait(credit_sems.at[kk], 1)
                rdma(kk, h).start()

        for kk in order:
            rdma(kk, N_HOPS - 1).wait_recv()
            store(kk, N_HOPS - 1)
            credit(kk)
            rdma(kk, N_HOPS - 2).wait_send()
            rdma(kk, N_HOPS - 1).wait_send()
            pl.semaphore_wait(credit_sems.at[kk], 2)

    return pl.pallas_call(
        body,
        out_shape=jax.ShapeDtypeStruct((m, n), jnp.float32),
        in_specs=[
            pl.BlockSpec(memory_space=pltpu.VMEM),
            pl.BlockSpec(memory_space=pltpu.VMEM),
        ],
        out_specs=pl.BlockSpec(memory_space=pltpu.VMEM),
        scratch_shapes=[
            pltpu.VMEM((N_STREAMS, 2, m // N_DEV, n // N_STREAMS),
                       jnp.float32),
            pltpu.VMEM((N_STREAMS, 2, m // N_DEV, n // N_STREAMS),
                       jnp.float32),
            pltpu.SemaphoreType.DMA((N_STREAMS, 2)),
            pltpu.SemaphoreType.DMA((N_STREAMS, 2)),
            pltpu.SemaphoreType.REGULAR((N_STREAMS,)),
        ],
        compiler_params=pltpu.CompilerParams(collective_id=0),
    )(A, B)


# device time: 114502 ns/iter; 2.2905x vs baseline; 1.2177x over previous
import jax
import jax.numpy as jnp
from jax import lax
from jax.experimental import pallas as pl
from jax.experimental.pallas import tpu as pltpu

N_DEV = 16
N_STREAMS = 4
N_HOPS = 2 * (N_DEV - 1)

_RING = (0, 4, 8, 12, 15, 11, 7, 3, 2, 6, 10, 14, 13, 9, 5, 1)
_POS = [0] * N_DEV
_NEXT = [0] * N_DEV
_PREV = [0] * N_DEV
for _p, _i in enumerate(_RING):
    _POS[_i] = _p
    _NEXT[_i] = _RING[(_p + 1) % N_DEV]
    _PREV[_i] = _RING[(_p - 1) % N_DEV]


def _sel(table, idx):
    out = jnp.int32(0)
    for i, v in enumerate(table):
        out = out + jnp.int32(v) * (idx == i).astype(jnp.int32)
    return out


def _gelu(z):
    return 0.5 * z * (1.0 + jnp.tanh(0.7978845608 * (z + 0.044715 * z * z * z)))


def kernel(A, B):
    m, k = A.shape
    k2, n = B.shape
    assert k == k2
    m_chunk = m // N_DEV
    n_stream = n // N_STREAMS

    def body(a_ref, b_ref, out_ref, send_buf, recv_buf, send_sems, recv_sems,
             credit_sems):
        my = lax.axis_index("i")
        pos = _sel(_POS, my)
        nxt = _sel(_NEXT, my)
        prv = _sel(_PREV, my)

        barrier_sem = pltpu.get_barrier_semaphore()
        for nbr in (prv, nxt):
            pl.semaphore_signal(
                barrier_sem, inc=1,
                device_id=(nbr,), device_id_type=pl.DeviceIdType.MESH,
            )
        pl.semaphore_wait(barrier_sem, 2)

        def rows(j):
            return pl.ds(j * m_chunk, m_chunk)

        def compute_chunk(j):
            out_ref[rows(j), :] = jnp.dot(
                a_ref[rows(j), :], b_ref[:, :],
                preferred_element_type=jnp.float32,
            )

        def is_f(kk):
            return kk < 2

        def cols(kk):
            return pl.ds(kk * n_stream, n_stream)

        def dst(kk):
            return nxt if is_f(kk) else prv

        def crd(kk):
            return prv if is_f(kk) else nxt

        def rs_send_chunk(kk, s):
            if is_f(kk):
                return lax.rem(pos + N_DEV - s, N_DEV)
            return lax.rem(pos + s, N_DEV)

        def own_chunk(kk):
            if is_f(kk):
                return lax.rem(pos + 1, N_DEV)
            return lax.rem(pos + N_DEV - 1, N_DEV)

        def ag_recv_chunk(kk, s):
            if is_f(kk):
                return lax.rem(pos + N_DEV - s, N_DEV)
            return lax.rem(pos + s, N_DEV)

        def rdma(kk, h):
            slot = h % 2
            return pltpu.make_async_remote_copy(
                src_ref=send_buf.at[kk, slot],
                dst_ref=recv_buf.at[kk, slot],
                send_sem=send_sems.at[kk, slot],
                recv_sem=recv_sems.at[kk, slot],
                device_id=(dst(kk),),
                device_id_type=pl.DeviceIdType.MESH,
            )

        def credit(kk):
            pl.semaphore_signal(
                credit_sems.at[kk], inc=1,
                device_id=(crd(kk),), device_id_type=pl.DeviceIdType.MESH,
            )

        def stage(kk, h):
            slot = h % 2
            if h == 0:
                send_buf[kk, slot, :, :] = out_ref[rows(rs_send_chunk(kk, 0)),
                                                   cols(kk)]
            elif h <= N_DEV - 2:
                send_buf[kk, slot, :, :] = (
                    recv_buf[kk, (h - 1) % 2, :, :]
                    + out_ref[rows(rs_send_chunk(kk, h)), cols(kk)]
                )
            elif h == N_DEV - 1:
                red = (recv_buf[kk, (h - 1) % 2, :, :]
                       + out_ref[rows(own_chunk(kk)), cols(kk)])
                g = _gelu(red)
                out_ref[rows(own_chunk(kk)), cols(kk)] = g
                send_buf[kk, slot, :, :] = g
            else:
                send_buf[kk, slot, :, :] = recv_buf[kk, (h - 1) % 2, :, :]

        def store(kk, h):
            if h >= N_DEV - 1:
                s = h - (N_DEV - 1)
                out_ref[rows(ag_recv_chunk(kk, s)), cols(kk)] = \
                    recv_buf[kk, h % 2, :, :]

        order = (0, 2, 1, 3)

        compute_chunk(pos)
        for kk in order:
            stage(kk, 0)
            rdma(kk, 0).start()

        for h in range(1, N_HOPS):
            if h <= N_DEV // 2:
                compute_chunk(lax.rem(pos + N_DEV - h, N_DEV))
                if h < N_DEV // 2:
                    compute_chunk(lax.rem(pos + h, N_DEV))
            for kk in order:
                if h >= 2:
                    rdma(kk, h).wait_send()
                rdma(kk, h - 1).wait_recv()
                store(kk, h - 1)
                stage(kk, h)
                credit(kk)
                if h >= 2:
                    pl.semaphore_wait(credit_sems.at[kk], 1)
                rdma(kk, h).start()

        for kk in order:
            rdma(kk, N_HOPS - 1).wait_recv()
            store(kk, N_HOPS - 1)
            credit(kk)
            rdma(kk, N_HOPS - 2).wait_send()
            rdma(kk, N_HOPS - 1).wait_send()
            pl.semaphore_wait(credit_sems.at[kk], 2)

    return pl.pallas_call(
        body,
        out_shape=jax.ShapeDtypeStruct((m, n), jnp.float32),
        in_specs=[
            pl.BlockSpec(memory_space=pltpu.VMEM),
            pl.BlockSpec(memory_space=pltpu.VMEM),
        ],
        out_specs=pl.BlockSpec(memory_space=pltpu.VMEM),
        scratch_shapes=[
            pltpu.VMEM((N_STREAMS, 2, m // N_DEV, n // N_STREAMS),
                       jnp.float32),
            pltpu.VMEM((N_STREAMS, 2, m // N_DEV, n // N_STREAMS),
                       jnp.float32),
            pltpu.SemaphoreType.DMA((N_STREAMS, 2)),
            pltpu.SemaphoreType.DMA((N_STREAMS, 2)),
            pltpu.SemaphoreType.REGULAR((N_STREAMS,)),
        ],
        compiler_params=pltpu.CompilerParams(collective_id=0),
    )(A, B)


# device time: 113951 ns/iter; 2.3016x vs baseline; 1.0048x over previous
import jax
import jax.numpy as jnp
from jax import lax
from jax.experimental import pallas as pl
from jax.experimental.pallas import tpu as pltpu

N_DEV = 16
N_HOPS = 2 * (N_DEV - 1)
N_CPD = 2
N_RPD = 2
N_STREAMS = 2 * N_CPD * N_RPD

_RING = (0, 4, 8, 12, 15, 11, 7, 3, 2, 6, 10, 14, 13, 9, 5, 1)
_POS = [0] * N_DEV
_NEXT = [0] * N_DEV
_PREV = [0] * N_DEV
for _p, _i in enumerate(_RING):
    _POS[_i] = _p
    _NEXT[_i] = _RING[(_p + 1) % N_DEV]
    _PREV[_i] = _RING[(_p - 1) % N_DEV]


def _sel(table, idx):
    out = jnp.int32(0)
    for i, v in enumerate(table):
        out = out + jnp.int32(v) * (idx == i).astype(jnp.int32)
    return out


def _gelu(z):
    return 0.5 * z * (1.0 + jnp.tanh(0.7978845608 * (z + 0.044715 * z * z * z)))


def kernel(A, B):
    m, k = A.shape
    k2, n = B.shape
    assert k == k2
    m_chunk = m // N_DEV
    m_sub = m_chunk // N_RPD
    n_sub = (n // 2) // N_CPD

    streams = []
    for c in range(N_CPD):
        for r in range(N_RPD):
            streams.append((True, c * n_sub, r * m_sub))
            streams.append((False, n // 2 + c * n_sub, r * m_sub))

    def body(a_ref, b_ref, out_ref, send_buf, recv_buf, send_sems, recv_sems,
             credit_sems):
        my = lax.axis_index("i")
        pos = _sel(_POS, my)
        nxt = _sel(_NEXT, my)
        prv = _sel(_PREV, my)

        barrier_sem = pltpu.get_barrier_semaphore()
        for nbr in (prv, nxt):
            pl.semaphore_signal(
                barrier_sem, inc=1,
                device_id=(nbr,), device_id_type=pl.DeviceIdType.MESH,
            )
        pl.semaphore_wait(barrier_sem, 2)

        def compute_chunk(j):
            out_ref[pl.ds(j * m_chunk, m_chunk), :] = jnp.dot(
                a_ref[pl.ds(j * m_chunk, m_chunk), :], b_ref[:, :],
                preferred_element_type=jnp.float32,
            )

        def tile(kk, j):
            fwd, col_off, row_off = streams[kk]
            return (pl.ds(j * m_chunk + row_off, m_sub),
                    pl.ds(col_off, n_sub))

        def dst(kk):
            return nxt if streams[kk][0] else prv

        def crd(kk):
            return prv if streams[kk][0] else nxt

        def rs_send_chunk(kk, s):
            if streams[kk][0]:
                return lax.rem(pos + N_DEV - s, N_DEV)
            return lax.rem(pos + s, N_DEV)

        def own_chunk(kk):
            if streams[kk][0]:
                return lax.rem(pos + 1, N_DEV)
            return lax.rem(pos + N_DEV - 1, N_DEV)

        def ag_recv_chunk(kk, s):
            if streams[kk][0]:
                return lax.rem(pos + N_DEV - s, N_DEV)
            return lax.rem(pos + s, N_DEV)

        def rdma(kk, h):
            slot = h % 2
            return pltpu.make_async_remote_copy(
                src_ref=send_buf.at[kk, slot],
                dst_ref=recv_buf.at[kk, slot],
                send_sem=send_sems.at[kk, slot],
                recv_sem=recv_sems.at[kk, slot],
                device_id=(dst(kk),),
                device_id_type=pl.DeviceIdType.MESH,
            )

        def credit(kk):
            pl.semaphore_signal(
                credit_sems.at[kk], inc=1,
                device_id=(crd(kk),), device_id_type=pl.DeviceIdType.MESH,
            )

        def stage(kk, h):
            slot = h % 2
            if h == 0:
                r, c = tile(kk, rs_send_chunk(kk, 0))
                send_buf[kk, slot, :, :] = out_ref[r, c]
            elif h <= N_DEV - 2:
                r, c = tile(kk, rs_send_chunk(kk, h))
                send_buf[kk, slot, :, :] = (
                    recv_buf[kk, (h - 1) % 2, :, :] + out_ref[r, c]
                )
            elif h == N_DEV - 1:
                r, c = tile(kk, own_chunk(kk))
                red = recv_buf[kk, (h - 1) % 2, :, :] + out_ref[r, c]
                g = _gelu(red)
                out_ref[r, c] = g
                send_buf[kk, slot, :, :] = g
            else:
                send_buf[kk, slot, :, :] = recv_buf[kk, (h - 1) % 2, :, :]

        def store(kk, h):
            if h >= N_DEV - 1:
                s = h - (N_DEV - 1)
                r, c = tile(kk, ag_recv_chunk(kk, s))
                out_ref[r, c] = recv_buf[kk, h % 2, :, :]

        order = range(N_STREAMS)

        compute_chunk(pos)
        for kk in order:
            stage(kk, 0)
            rdma(kk, 0).start()

        for h in range(1, N_HOPS):
            if h <= N_DEV // 2:
                compute_chunk(lax.rem(pos + N_DEV - h, N_DEV))
                if h < N_DEV // 2:
                    compute_chunk(lax.rem(pos + h, N_DEV))
            for kk in order:
                if h >= 2:
                    rdma(kk, h).wait_send()
                rdma(kk, h - 1).wait_recv()
                store(kk, h - 1)
                stage(kk, h)
                credit(kk)
                if h >= 2:
                    pl.semaphore_wait(credit_sems.at[kk], 1)
                rdma(kk, h).start()

        for kk in order:
            rdma(kk, N_HOPS - 1).wait_recv()
            store(kk, N_HOPS - 1)
            credit(kk)
            rdma(kk, N_HOPS - 2).wait_send()
            rdma(kk, N_HOPS - 1).wait_send()
            pl.semaphore_wait(credit_sems.at[kk], 2)

    m_sub_ = m // N_DEV // N_RPD
    n_sub_ = n // 2 // N_CPD
    return pl.pallas_call(
        body,
        out_shape=jax.ShapeDtypeStruct((m, n), jnp.float32),
        in_specs=[
            pl.BlockSpec(memory_space=pltpu.VMEM),
            pl.BlockSpec(memory_space=pltpu.VMEM),
        ],
        out_specs=pl.BlockSpec(memory_space=pltpu.VMEM),
        scratch_shapes=[
            pltpu.VMEM((N_STREAMS, 2, m_sub_, n_sub_), jnp.float32),
            pltpu.VMEM((N_STREAMS, 2, m_sub_, n_sub_), jnp.float32),
            pltpu.SemaphoreType.DMA((N_STREAMS, 2)),
            pltpu.SemaphoreType.DMA((N_STREAMS, 2)),
            pltpu.SemaphoreType.REGULAR((N_STREAMS,)),
        ],
        compiler_params=pltpu.CompilerParams(collective_id=0),
    )(A, B)
